# Initial kernel scaffold; baseline (speedup 1.0000x reference)
import jax
import jax.numpy as jnp
from jax.experimental import pallas as pl


def kernel(x, edge_index, Wn0, Ws0, b0, Wn1, Ws1, b1, Wn2, Ws2, b2, Wn3, Ws3, b3, Wn4, Ws4, b4, Wn5, Ws5, b5):
    raise NotImplementedError("write your pallas kernel here")


# speedup vs baseline: 7.9211x; 7.9211x over previous
import functools

import jax
import jax.numpy as jnp
from jax import lax
from jax.experimental import pallas as pl
from jax.experimental.pallas import tpu as pltpu
from jax.experimental.pallas import tpu_sc as plsc

F32 = jnp.float32
I32 = jnp.int32

NC = 2
NS = 16
LN = 128
CH = 8


def _make_segsum(n_nodes, n_rows128, quad):
    rows_node_pt = -(-n_nodes // NS)
    rows_node_pt = ((rows_node_pt + 7) // 8) * 8
    NP = NS * rows_node_pt
    ZB = rows_node_pt // 8
    n_zero = rows_node_pt // ZB
    if quad:
        rows_per_tile = n_rows128 // NS
    else:
        rows_per_tile = n_rows128 // (NC * NS)
    nchunks = rows_per_tile // CH
    n_out = 4 if quad else NC

    mesh = plsc.VectorSubcoreMesh(core_axis_name="c", subcore_axis_name="s")

    @functools.partial(
        pl.kernel,
        out_type=jax.ShapeDtypeStruct((n_out, NP, 16), F32),
        mesh=mesh,
        scratch_types=[
            pltpu.VMEM((CH, LN), I32),
            pltpu.VMEM((CH, LN), I32),
            pltpu.VMEM((CH, LN, 16), F32),
            pltpu.VMEM((ZB, 16), F32),
            pltpu.VMEM_SHARED((NP, 16), F32),
            pltpu.SemaphoreType.DMA,
        ],
        compiler_params=pltpu.CompilerParams(use_tc_tiling_on_sc=False),
    )
    def k(table, idx2, dst2, out, idx_v, dst_v, rows_v, zb, acc, sem):
        c = lax.axis_index("c")
        s = lax.axis_index("s")

        def zb_body(i, carry):
            zb[i, :] = jnp.zeros((16,), F32)
            return carry

        lax.fori_loop(0, ZB, zb_body, 0)
        r0 = s * rows_node_pt

        def one_pass(q, row_base):
            for z in range(n_zero):
                pltpu.sync_copy(zb, acc.at[pl.ds(r0 + z * ZB, ZB)])
            plsc.subcore_barrier()

            def chunk(i, carry):
                rb = row_base + i * CH
                pltpu.sync_copy(idx2.at[pl.ds(rb, CH)], idx_v)
                pltpu.sync_copy(dst2.at[pl.ds(rb, CH)], dst_v)
                if quad:
                    for j in range(CH):
                        for v in range(LN // 16):
                            t = idx_v[j, pl.ds(v * 16, 16)]
                            idx_v[j, pl.ds(v * 16, 16)] = t * 4 + q
                descs = [
                    pltpu.async_copy(table.at[idx_v.at[j]], rows_v.at[j], sem)
                    for j in range(CH)
                ]
                for d in descs:
                    d.wait()
                for j in range(CH):
                    pltpu.sync_copy(rows_v.at[j], acc.at[dst_v.at[j]], add=True)
                return carry

            lax.fori_loop(0, nchunks, chunk, 0)
            plsc.subcore_barrier()

        if quad:
            for qq in range(2):
                q = c * 2 + qq
                one_pass(q, s * rows_per_tile)
                pltpu.sync_copy(
                    acc.at[pl.ds(r0, rows_node_pt)],
                    out.at[q, pl.ds(r0, rows_node_pt)],
                )
                if qq == 0:
                    plsc.subcore_barrier()
        else:
            one_pass(None, (c * NS + s) * rows_per_tile)
            pltpu.sync_copy(
                acc.at[pl.ds(r0, rows_node_pt)],
                out.at[c, pl.ds(r0, rows_node_pt)],
            )

    return k


def _tc_layer0(n, bn):
    grid = (n // bn,)

    def body(accp, x16, wn, ws, b, h_out, cinv_out):
        acc = accp[0] + accp[1]
        cinv = 1.0 / jnp.maximum(acc[:, 6:7], 1.0)
        mean = acc * cinv
        z = (jnp.dot(mean, wn[...], preferred_element_type=F32)
             + jnp.dot(x16[...], ws[...], preferred_element_type=F32)
             + b[...])
        h_out[...] = jnp.maximum(z, 0.0)
        cinv_out[...] = jnp.broadcast_to(cinv, (bn, 16))

    return pl.pallas_call(
        body,
        grid=grid,
        in_specs=[
            pl.BlockSpec((2, bn, 16), lambda i: (0, i, 0)),
            pl.BlockSpec((bn, 16), lambda i: (i, 0)),
            pl.BlockSpec((16, 64), lambda i: (0, 0)),
            pl.BlockSpec((16, 64), lambda i: (0, 0)),
            pl.BlockSpec((1, 64), lambda i: (0, 0)),
        ],
        out_specs=[
            pl.BlockSpec((bn, 64), lambda i: (i, 0)),
            pl.BlockSpec((bn, 16), lambda i: (i, 0)),
        ],
        out_shape=[
            jax.ShapeDtypeStruct((n, 64), F32),
            jax.ShapeDtypeStruct((n, 16), F32),
        ],
    )


def _tc_mid(n, bn, last):
    grid = (n // bn,)

    def body(agg, h, cinv, wn, ws, b, *rest):
        if last:
            w5, b5, out = rest
        else:
            (out,) = rest
        a = jnp.concatenate([agg[0], agg[1], agg[2], agg[3]], axis=1)
        mean = a * cinv[:, 0:1]
        z = (jnp.dot(mean, wn[...], preferred_element_type=F32)
             + jnp.dot(h[...], ws[...], preferred_element_type=F32)
             + b[...])
        h_new = jnp.maximum(z, 0.0) + h[...]
        if last:
            out[...] = (jnp.dot(h_new, w5[...], preferred_element_type=F32)
                        + b5[...])
        else:
            out[...] = h_new

    in_specs = [
        pl.BlockSpec((4, bn, 16), lambda i: (0, i, 0)),
        pl.BlockSpec((bn, 64), lambda i: (i, 0)),
        pl.BlockSpec((bn, 16), lambda i: (i, 0)),
        pl.BlockSpec((64, 64), lambda i: (0, 0)),
        pl.BlockSpec((64, 64), lambda i: (0, 0)),
        pl.BlockSpec((1, 64), lambda i: (0, 0)),
    ]
    if last:
        in_specs += [
            pl.BlockSpec((64, 16), lambda i: (0, 0)),
            pl.BlockSpec((1, 16), lambda i: (0, 0)),
        ]
        out_spec = pl.BlockSpec((bn, 16), lambda i: (i, 0))
        out_shape = jax.ShapeDtypeStruct((n, 16), F32)
    else:
        out_spec = pl.BlockSpec((bn, 64), lambda i: (i, 0))
        out_shape = jax.ShapeDtypeStruct((n, 64), F32)

    return pl.pallas_call(
        body, grid=grid, in_specs=in_specs, out_specs=out_spec,
        out_shape=out_shape,
    )


def _tc_final(n, bn):
    grid = (n // bn,)

    def body(accp, ts, cinv, out):
        acc0 = accp[0, :, 0:1] + accp[1, :, 0:1]
        out[...] = acc0 * cinv[:, 0:1] + ts[:, 1:2]

    return pl.pallas_call(
        body,
        grid=grid,
        in_specs=[
            pl.BlockSpec((2, bn, 16), lambda i: (0, i, 0)),
            pl.BlockSpec((bn, 16), lambda i: (i, 0)),
            pl.BlockSpec((bn, 16), lambda i: (i, 0)),
        ],
        out_specs=pl.BlockSpec((bn, 1), lambda i: (i, 0)),
        out_shape=jax.ShapeDtypeStruct((n, 1), F32),
    )


def kernel(x, edge_index, Wn0, Ws0, b0, Wn1, Ws1, b1, Wn2, Ws2, b2,
           Wn3, Ws3, b3, Wn4, Ws4, b4, Wn5, Ws5, b5):
    n, in_dim = x.shape
    e = edge_index.shape[1]

    src = edge_index[0].astype(I32)
    dst = edge_index[1].astype(I32)
    unit = NC * NS * CH * LN
    epad = ((e + unit - 1) // unit) * unit
    pad = epad - e
    src_p = jnp.concatenate([src, jnp.zeros((pad,), I32)])
    dst_p = jnp.concatenate([dst, jnp.full((pad,), n, I32)])
    idx2 = src_p.reshape(-1, LN)
    dst2 = dst_p.reshape(-1, LN)
    n_rows128 = epad // LN

    x16 = jnp.concatenate(
        [x, jnp.ones((n, 1), F32), jnp.zeros((n, 16 - in_dim - 1), F32)], 1)
    wn0p = jnp.zeros((16, 64), F32).at[:in_dim].set(Wn0)
    ws0p = jnp.zeros((16, 64), F32).at[:in_dim].set(Ws0)
    w5p = jnp.zeros((64, 16), F32).at[:, 0:1].set(Wn5).at[:, 1:2].set(Ws5)
    b5p = jnp.zeros((1, 16), F32).at[0, 1].set(b5[0])

    seg_single = _make_segsum(n, n_rows128, quad=False)
    seg_quad = _make_segsum(n, n_rows128, quad=True)
    bn = 1000

    accp0 = seg_single(x16, idx2, dst2)
    h, cinv = _tc_layer0(n, bn)(accp0, x16, wn0p, ws0p, b0.reshape(1, -1))

    for (wn, ws, b) in ((Wn1, Ws1, b1), (Wn2, Ws2, b2), (Wn3, Ws3, b3)):
        agg = seg_quad(h.reshape(4 * n, 16), idx2, dst2)
        h = _tc_mid(n, bn, last=False)(
            agg, h, cinv, wn, ws, b.reshape(1, -1))

    agg = seg_quad(h.reshape(4 * n, 16), idx2, dst2)
    ts = _tc_mid(n, bn, last=True)(
        agg, h, cinv, Wn4, Ws4, b4.reshape(1, -1), w5p, b5p)

    accp5 = seg_single(ts, idx2, dst2)
    out = _tc_final(n, bn)(accp5, ts, cinv)
    return out[:, 0]


# speedup vs baseline: 10.0844x; 1.2731x over previous
import functools

import jax
import jax.numpy as jnp
from jax import lax
from jax.experimental import pallas as pl
from jax.experimental.pallas import tpu as pltpu
from jax.experimental.pallas import tpu_sc as plsc

F32 = jnp.float32
I32 = jnp.int32

NC = 2
NS = 16
LN = 128
CH = 8


EB = 512


def _make_segsum(n_nodes, epad, quad):
    rows_node_pt = -(-n_nodes // NS)
    rows_node_pt = ((rows_node_pt + 7) // 8) * 8
    NP = NS * rows_node_pt
    ZB = rows_node_pt // 16
    n_zero = rows_node_pt // ZB
    if quad:
        edges_pt = epad // NS
    else:
        edges_pt = epad // (NC * NS)
    nblocks = edges_pt // EB
    max_eb = epad - EB
    n_out = 4 if quad else NC

    mesh = plsc.VectorSubcoreMesh(core_axis_name="c", subcore_axis_name="s")

    @functools.partial(
        pl.kernel,
        out_type=jax.ShapeDtypeStruct((n_out, NP, 16), F32),
        mesh=mesh,
        scratch_types=[
            pltpu.VMEM((3, EB), I32),
            pltpu.VMEM((3, EB), I32),
            pltpu.VMEM((2, EB, 16), F32),
            pltpu.VMEM((ZB, 16), F32),
            pltpu.VMEM_SHARED((NP, 16), F32),
            pltpu.SemaphoreType.DMA,
            pltpu.SemaphoreType.DMA,
            pltpu.SemaphoreType.DMA,
        ],
        compiler_params=pltpu.CompilerParams(use_tc_tiling_on_sc=False),
    )
    def k(table, idx1, dst1, out, idx_v, dst_v, rows_v, zb, acc,
          lsem, gsem, ssem):
        c = lax.axis_index("c")
        s = lax.axis_index("s")

        def zb_body(i, carry):
            zb[i, :] = jnp.zeros((16,), F32)
            return carry

        lax.fori_loop(0, ZB, zb_body, 0)
        r0 = s * rows_node_pt

        def one_pass(q, edge_base):
            for z in range(n_zero):
                pltpu.async_copy(zb, acc.at[pl.ds(r0 + z * ZB, ZB)], gsem)
            for z in range(n_zero):
                pltpu.make_async_copy(
                    zb, acc.at[pl.ds(r0 + z * ZB, ZB)], gsem).wait()
            plsc.subcore_barrier()

            pltpu.async_copy(idx1.at[pl.ds(edge_base, EB)], idx_v.at[0], lsem)
            pltpu.async_copy(dst1.at[pl.ds(edge_base, EB)], dst_v.at[0], lsem)

            def block(i, carry):
                p = i % 2
                cp = i % 3
                np_ = (i + 1) % 3
                eb_next = jnp.minimum(edge_base + (i + 1) * EB, max_eb)
                pltpu.make_async_copy(
                    idx1.at[pl.ds(edge_base, EB)], idx_v.at[cp], lsem).wait()
                pltpu.make_async_copy(
                    dst1.at[pl.ds(edge_base, EB)], dst_v.at[cp], lsem).wait()
                pltpu.async_copy(
                    idx1.at[pl.ds(eb_next, EB)], idx_v.at[np_], lsem)
                pltpu.async_copy(
                    dst1.at[pl.ds(eb_next, EB)], dst_v.at[np_], lsem)
                if quad:
                    for v in range(EB // 16):
                        t = idx_v[cp, pl.ds(v * 16, 16)]
                        idx_v[cp, pl.ds(v * 16, 16)] = t * 4 + q
                g = pltpu.async_copy(
                    table.at[idx_v.at[cp]], rows_v.at[p], gsem)
                @pl.when(i > 0)
                def _():
                    pltpu.make_async_copy(
                        rows_v.at[1 - p], acc.at[pl.ds(0, EB)], ssem).wait()
                g.wait()
                pltpu.async_copy(
                    rows_v.at[p], acc.at[dst_v.at[cp]], ssem, add=True)
                return carry

            lax.fori_loop(0, nblocks, block, 0)
            pltpu.make_async_copy(
                rows_v.at[0], acc.at[pl.ds(0, EB)], ssem).wait()
            pltpu.make_async_copy(
                idx1.at[pl.ds(0, EB)], idx_v.at[0], lsem).wait()
            pltpu.make_async_copy(
                dst1.at[pl.ds(0, EB)], dst_v.at[0], lsem).wait()
            plsc.subcore_barrier()

        if quad:
            for qq in range(2):
                q = c * 2 + qq
                one_pass(q, s * edges_pt)
                pltpu.sync_copy(
                    acc.at[pl.ds(r0, rows_node_pt)],
                    out.at[q, pl.ds(r0, rows_node_pt)],
                )
                if qq == 0:
                    plsc.subcore_barrier()
        else:
            one_pass(None, (c * NS + s) * edges_pt)
            pltpu.sync_copy(
                acc.at[pl.ds(r0, rows_node_pt)],
                out.at[c, pl.ds(r0, rows_node_pt)],
            )

    return k


def _tc_layer0(n, bn):
    grid = (n // bn,)

    def body(accp, x16, wn, ws, b, h_out, cinv_out):
        acc = accp[0] + accp[1]
        cinv = 1.0 / jnp.maximum(acc[:, 6:7], 1.0)
        mean = acc * cinv
        z = (jnp.dot(mean, wn[...], preferred_element_type=F32)
             + jnp.dot(x16[...], ws[...], preferred_element_type=F32)
             + b[...])
        h_out[...] = jnp.maximum(z, 0.0)
        cinv_out[...] = jnp.broadcast_to(cinv, (bn, 16))

    return pl.pallas_call(
        body,
        grid=grid,
        in_specs=[
            pl.BlockSpec((2, bn, 16), lambda i: (0, i, 0)),
            pl.BlockSpec((bn, 16), lambda i: (i, 0)),
            pl.BlockSpec((16, 64), lambda i: (0, 0)),
            pl.BlockSpec((16, 64), lambda i: (0, 0)),
            pl.BlockSpec((1, 64), lambda i: (0, 0)),
        ],
        out_specs=[
            pl.BlockSpec((bn, 64), lambda i: (i, 0)),
            pl.BlockSpec((bn, 16), lambda i: (i, 0)),
        ],
        out_shape=[
            jax.ShapeDtypeStruct((n, 64), F32),
            jax.ShapeDtypeStruct((n, 16), F32),
        ],
    )


def _tc_mid(n, bn, last):
    grid = (n // bn,)

    def body(agg, h, cinv, wn, ws, b, *rest):
        if last:
            w5, b5, out = rest
        else:
            (out,) = rest
        a = jnp.concatenate([agg[0], agg[1], agg[2], agg[3]], axis=1)
        mean = a * cinv[:, 0:1]
        z = (jnp.dot(mean, wn[...], preferred_element_type=F32)
             + jnp.dot(h[...], ws[...], preferred_element_type=F32)
             + b[...])
        h_new = jnp.maximum(z, 0.0) + h[...]
        if last:
            out[...] = (jnp.dot(h_new, w5[...], preferred_element_type=F32)
                        + b5[...])
        else:
            out[...] = h_new

    in_specs = [
        pl.BlockSpec((4, bn, 16), lambda i: (0, i, 0)),
        pl.BlockSpec((bn, 64), lambda i: (i, 0)),
        pl.BlockSpec((bn, 16), lambda i: (i, 0)),
        pl.BlockSpec((64, 64), lambda i: (0, 0)),
        pl.BlockSpec((64, 64), lambda i: (0, 0)),
        pl.BlockSpec((1, 64), lambda i: (0, 0)),
    ]
    if last:
        in_specs += [
            pl.BlockSpec((64, 16), lambda i: (0, 0)),
            pl.BlockSpec((1, 16), lambda i: (0, 0)),
        ]
        out_spec = pl.BlockSpec((bn, 16), lambda i: (i, 0))
        out_shape = jax.ShapeDtypeStruct((n, 16), F32)
    else:
        out_spec = pl.BlockSpec((bn, 64), lambda i: (i, 0))
        out_shape = jax.ShapeDtypeStruct((n, 64), F32)

    return pl.pallas_call(
        body, grid=grid, in_specs=in_specs, out_specs=out_spec,
        out_shape=out_shape,
    )


def _tc_final(n, bn):
    grid = (n // bn,)

    def body(accp, ts, cinv, out):
        acc0 = accp[0, :, 0:1] + accp[1, :, 0:1]
        out[...] = acc0 * cinv[:, 0:1] + ts[:, 1:2]

    return pl.pallas_call(
        body,
        grid=grid,
        in_specs=[
            pl.BlockSpec((2, bn, 16), lambda i: (0, i, 0)),
            pl.BlockSpec((bn, 16), lambda i: (i, 0)),
            pl.BlockSpec((bn, 16), lambda i: (i, 0)),
        ],
        out_specs=pl.BlockSpec((bn, 1), lambda i: (i, 0)),
        out_shape=jax.ShapeDtypeStruct((n, 1), F32),
    )


def kernel(x, edge_index, Wn0, Ws0, b0, Wn1, Ws1, b1, Wn2, Ws2, b2,
           Wn3, Ws3, b3, Wn4, Ws4, b4, Wn5, Ws5, b5):
    n, in_dim = x.shape
    e = edge_index.shape[1]

    src = edge_index[0].astype(I32)
    dst = edge_index[1].astype(I32)
    unit = NC * NS * CH * LN
    epad = ((e + unit - 1) // unit) * unit
    pad = epad - e
    src_p = jnp.concatenate([src, jnp.zeros((pad,), I32)])
    dst_p = jnp.concatenate([dst, jnp.full((pad,), n, I32)])

    x16 = jnp.concatenate(
        [x, jnp.ones((n, 1), F32), jnp.zeros((n, 16 - in_dim - 1), F32)], 1)
    wn0p = jnp.zeros((16, 64), F32).at[:in_dim].set(Wn0)
    ws0p = jnp.zeros((16, 64), F32).at[:in_dim].set(Ws0)
    w5p = jnp.zeros((64, 16), F32).at[:, 0:1].set(Wn5).at[:, 1:2].set(Ws5)
    b5p = jnp.zeros((1, 16), F32).at[0, 1].set(b5[0])

    seg_single = _make_segsum(n, epad, quad=False)
    seg_quad = _make_segsum(n, epad, quad=True)
    bn = 1000

    accp0 = seg_single(x16, src_p, dst_p)
    h, cinv = _tc_layer0(n, bn)(accp0, x16, wn0p, ws0p, b0.reshape(1, -1))

    for (wn, ws, b) in ((Wn1, Ws1, b1), (Wn2, Ws2, b2), (Wn3, Ws3, b3)):
        agg = seg_quad(h.reshape(4 * n, 16), src_p, dst_p)
        h = _tc_mid(n, bn, last=False)(
            agg, h, cinv, wn, ws, b.reshape(1, -1))

    agg = seg_quad(h.reshape(4 * n, 16), src_p, dst_p)
    ts = _tc_mid(n, bn, last=True)(
        agg, h, cinv, Wn4, Ws4, b4.reshape(1, -1), w5p, b5p)

    accp5 = seg_single(ts, src_p, dst_p)
    out = _tc_final(n, bn)(accp5, ts, cinv)
    return out[:, 0]


# speedup vs baseline: 14.5136x; 1.4392x over previous
import functools

import jax
import jax.numpy as jnp
from jax import lax
from jax.experimental import pallas as pl
from jax.experimental.pallas import tpu as pltpu
from jax.experimental.pallas import tpu_sc as plsc

F32 = jnp.float32
I32 = jnp.int32

NC = 2
NS = 16
EB = 512


def _make_segsum(n_pad, epad, nq):
    NP = n_pad
    rows_node_pt = NP // NS
    ZB = rows_node_pt // 16
    n_zero = rows_node_pt // ZB
    if nq == 4:
        edges_pt = epad // NS
    else:
        edges_pt = epad // (NC * NS)
    nblocks = edges_pt // EB
    max_eb = epad - EB
    n_out = 4 if nq == 4 else NC

    mesh = plsc.VectorSubcoreMesh(core_axis_name="c", subcore_axis_name="s")

    @functools.partial(
        pl.kernel,
        out_type=jax.ShapeDtypeStruct((n_out, NP, 16), F32),
        mesh=mesh,
        scratch_types=[
            pltpu.VMEM((3, EB), I32),
            pltpu.VMEM((3, EB), I32),
            pltpu.VMEM((2, EB, 16), F32),
            pltpu.VMEM((ZB, 16), F32),
            pltpu.VMEM_SHARED((NP, 16), F32),
            pltpu.SemaphoreType.DMA,
            pltpu.SemaphoreType.DMA,
            pltpu.SemaphoreType.DMA,
        ],
        compiler_params=pltpu.CompilerParams(use_tc_tiling_on_sc=False),
    )
    def k(table, idx1, dst1, out, idx_v, dst_v, rows_v, zb, acc,
          lsem, gsem, ssem):
        c = lax.axis_index("c")
        s = lax.axis_index("s")

        def zb_body(i, carry):
            zb[i, :] = jnp.zeros((16,), F32)
            return carry

        lax.fori_loop(0, ZB, zb_body, 0)
        r0 = s * rows_node_pt

        def one_pass(q, edge_base):
            for z in range(n_zero):
                pltpu.async_copy(zb, acc.at[pl.ds(r0 + z * ZB, ZB)], gsem)
            for z in range(n_zero):
                pltpu.make_async_copy(
                    zb, acc.at[pl.ds(r0 + z * ZB, ZB)], gsem).wait()
            plsc.subcore_barrier()

            pltpu.async_copy(idx1.at[pl.ds(edge_base, EB)], idx_v.at[0], lsem)
            pltpu.async_copy(dst1.at[pl.ds(edge_base, EB)], dst_v.at[0], lsem)

            def block(i, carry):
                p = i % 2
                cp = i % 3
                np_ = (i + 1) % 3
                eb_next = jnp.minimum(edge_base + (i + 1) * EB, max_eb)
                pltpu.make_async_copy(
                    idx1.at[pl.ds(edge_base, EB)], idx_v.at[cp], lsem).wait()
                pltpu.make_async_copy(
                    dst1.at[pl.ds(edge_base, EB)], dst_v.at[cp], lsem).wait()
                pltpu.async_copy(
                    idx1.at[pl.ds(eb_next, EB)], idx_v.at[np_], lsem)
                pltpu.async_copy(
                    dst1.at[pl.ds(eb_next, EB)], dst_v.at[np_], lsem)
                g = pltpu.async_copy(
                    table.at[q].at[idx_v.at[cp]], rows_v.at[p], gsem)
                @pl.when(i > 0)
                def _():
                    pltpu.make_async_copy(
                        rows_v.at[1 - p], acc.at[pl.ds(0, EB)], ssem).wait()
                g.wait()
                pltpu.async_copy(
                    rows_v.at[p], acc.at[dst_v.at[cp]], ssem, add=True)
                return carry

            lax.fori_loop(0, nblocks, block, 0)
            pltpu.make_async_copy(
                rows_v.at[0], acc.at[pl.ds(0, EB)], ssem).wait()
            pltpu.make_async_copy(
                idx1.at[pl.ds(0, EB)], idx_v.at[0], lsem).wait()
            pltpu.make_async_copy(
                dst1.at[pl.ds(0, EB)], dst_v.at[0], lsem).wait()
            plsc.subcore_barrier()

        if nq == 4:
            for qq in range(2):
                q = c * 2 + qq
                one_pass(q, s * edges_pt)
                pltpu.sync_copy(
                    acc.at[pl.ds(r0, rows_node_pt)],
                    out.at[q, pl.ds(r0, rows_node_pt)],
                )
                if qq == 0:
                    plsc.subcore_barrier()
        else:
            one_pass(0, (c * NS + s) * edges_pt)
            pltpu.sync_copy(
                acc.at[pl.ds(r0, rows_node_pt)],
                out.at[c, pl.ds(r0, rows_node_pt)],
            )

    return k


def _dot(a, b):
    return jnp.dot(a, b, preferred_element_type=F32)


def _tc_layer0(n_pad, bn):
    grid = (n_pad // bn,)
    br = bn // 8

    def body(accp, x8, wn, ws, b, s6, h_out, cinv_out):
        a8 = accp[0] + accp[1]
        cnt = _dot(a8, s6[...])
        cinv = 1.0 / jnp.maximum(cnt, 1.0)
        z = _dot(a8 * cinv, wn[...]) + _dot(x8[...], ws[...]) + b[...]
        hc = jnp.maximum(z, 0.0)
        for qq in range(4):
            h_out[qq, :, :] = hc[:, 128 * qq:128 * (qq + 1)]
        cinv_out[...] = cinv

    return pl.pallas_call(
        body,
        grid=grid,
        in_specs=[
            pl.BlockSpec((2, br, 128), lambda i: (0, i, 0)),
            pl.BlockSpec((br, 128), lambda i: (i, 0)),
            pl.BlockSpec((128, 512), lambda i: (0, 0)),
            pl.BlockSpec((128, 512), lambda i: (0, 0)),
            pl.BlockSpec((1, 512), lambda i: (0, 0)),
            pl.BlockSpec((128, 128), lambda i: (0, 0)),
        ],
        out_specs=[
            pl.BlockSpec((4, br, 128), lambda i: (0, i, 0)),
            pl.BlockSpec((br, 128), lambda i: (i, 0)),
        ],
        out_shape=[
            jax.ShapeDtypeStruct((4, n_pad // 8, 128), F32),
            jax.ShapeDtypeStruct((n_pad // 8, 128), F32),
        ],
    )


def _tc_mid(n_pad, bn, last):
    grid = (n_pad // bn,)
    br = bn // 8

    def body(agg, h, cinv, wn, ws, b, *rest):
        if last:
            w5, b5, out = rest
        else:
            (out,) = rest
        x = jnp.concatenate([agg[qq] for qq in range(4)], axis=1)
        hh = jnp.concatenate([h[qq] for qq in range(4)], axis=1)
        ci = jnp.concatenate([cinv[...]] * 4, axis=1)
        z = _dot(x * ci, wn[...]) + _dot(hh, ws[...]) + b[...]
        hn = jnp.maximum(z, 0.0) + hh
        if last:
            out[...] = _dot(hn, w5[...]) + b5[...]
        else:
            for qq in range(4):
                out[qq, :, :] = hn[:, 128 * qq:128 * (qq + 1)]

    in_specs = [
        pl.BlockSpec((4, br, 128), lambda i: (0, i, 0)),
        pl.BlockSpec((4, br, 128), lambda i: (0, i, 0)),
        pl.BlockSpec((br, 128), lambda i: (i, 0)),
        pl.BlockSpec((512, 512), lambda i: (0, 0)),
        pl.BlockSpec((512, 512), lambda i: (0, 0)),
        pl.BlockSpec((1, 512), lambda i: (0, 0)),
    ]
    if last:
        in_specs += [
            pl.BlockSpec((512, 128), lambda i: (0, 0)),
            pl.BlockSpec((1, 128), lambda i: (0, 0)),
        ]
        out_spec = pl.BlockSpec((br, 128), lambda i: (i, 0))
        out_shape = jax.ShapeDtypeStruct((n_pad // 8, 128), F32)
    else:
        out_spec = pl.BlockSpec((4, br, 128), lambda i: (0, i, 0))
        out_shape = jax.ShapeDtypeStruct((4, n_pad // 8, 128), F32)

    return pl.pallas_call(
        body, grid=grid, in_specs=in_specs, out_specs=out_spec,
        out_shape=out_shape,
    )


def _tc_final(n_pad, bn):
    grid = (n_pad // bn,)
    br = bn // 8

    def body(accp, ts, cinv, mt, ms, out):
        op = accp[0] + accp[1]
        out[...] = _dot(op * cinv[...], mt[...]) + _dot(ts[...], ms[...])

    return pl.pallas_call(
        body,
        grid=grid,
        in_specs=[
            pl.BlockSpec((2, br, 128), lambda i: (0, i, 0)),
            pl.BlockSpec((br, 128), lambda i: (i, 0)),
            pl.BlockSpec((br, 128), lambda i: (i, 0)),
            pl.BlockSpec((128, 128), lambda i: (0, 0)),
            pl.BlockSpec((128, 128), lambda i: (0, 0)),
        ],
        out_specs=pl.BlockSpec((br, 128), lambda i: (i, 0)),
        out_shape=jax.ShapeDtypeStruct((n_pad // 8, 128), F32),
    )


def kernel(x, edge_index, Wn0, Ws0, b0, Wn1, Ws1, b1, Wn2, Ws2, b2,
           Wn3, Ws3, b3, Wn4, Ws4, b4, Wn5, Ws5, b5):
    n, in_dim = x.shape
    e = edge_index.shape[1]

    bn = 2048
    NP = -(-n // (NS * 8)) * (NS * 8)
    while NP % bn:
        NP += NS * 8

    src = edge_index[0].astype(I32)
    dst = edge_index[1].astype(I32)
    unit = NC * NS * EB
    epad = ((e + unit - 1) // unit) * unit
    pad = epad - e
    src_p = jnp.concatenate([src, jnp.zeros((pad,), I32)])
    dst_p = jnp.concatenate([dst, jnp.full((pad,), n, I32)])

    x16 = jnp.concatenate(
        [x, jnp.ones((n, 1), F32), jnp.zeros((n, 16 - in_dim - 1), F32)], 1)
    x16 = jnp.concatenate([x16, jnp.zeros((NP - n, 16), F32)], 0)
    x8 = x16.reshape(NP // 8, 128)

    eye8 = jnp.eye(8, dtype=F32)
    wn0p = jnp.zeros((16, 64), F32).at[:in_dim].set(Wn0)
    ws0p = jnp.zeros((16, 64), F32).at[:in_dim].set(Ws0)

    def big0(w):
        w4 = w.reshape(16, 4, 16)
        return jnp.einsum('cpd,jk->jcpkd', w4, eye8).reshape(128, 512)

    def big(w):
        w4 = w.reshape(4, 16, 4, 16)
        return jnp.einsum('qcpd,jk->qjcpkd', w4, eye8).reshape(512, 512)

    def bpack(b):
        return jnp.tile(b.reshape(4, 1, 16), (1, 8, 1)).reshape(1, 512)

    s6 = jnp.kron(eye8, jnp.zeros((16, 16), F32).at[6, :].set(1.0))
    mt = jnp.kron(eye8, jnp.zeros((16, 16), F32).at[0, 0].set(1.0))
    ms = jnp.kron(eye8, jnp.zeros((16, 16), F32).at[1, 0].set(1.0))

    w5p = jnp.zeros((64, 16), F32).at[:, 0:1].set(Wn5).at[:, 1:2].set(Ws5)
    w5big = jnp.einsum('qcd,jk->qjckd', w5p.reshape(4, 16, 16),
                       eye8).reshape(512, 128)
    b5a = jnp.tile(jnp.zeros((1, 16), F32).at[0, 1].set(b5[0]),
                   (1, 8)).reshape(1, 128)

    seg_single = _make_segsum(NP, epad, nq=1)
    seg_quad = _make_segsum(NP, epad, nq=4)

    def to128(a):
        return a.reshape(a.shape[0], NP // 8, 128)

    accp0 = to128(seg_single(x16.reshape(1, NP, 16), src_p, dst_p))
    h, cinv = _tc_layer0(NP, bn)(
        accp0, x8, big0(wn0p), big0(ws0p), bpack(b0), s6)

    for (wn, ws, b) in ((Wn1, Ws1, b1), (Wn2, Ws2, b2), (Wn3, Ws3, b3)):
        agg = to128(seg_quad(h.reshape(4, NP, 16), src_p, dst_p))
        h = _tc_mid(NP, bn, last=False)(
            agg, h, cinv, big(wn), big(ws), bpack(b))

    agg = to128(seg_quad(h.reshape(4, NP, 16), src_p, dst_p))
    ts = _tc_mid(NP, bn, last=True)(
        agg, h, cinv, big(Wn4), big(Ws4), bpack(b4), w5big, b5a)

    accp5 = to128(seg_single(ts.reshape(1, NP, 16), src_p, dst_p))
    out = _tc_final(NP, bn)(accp5, ts, cinv, mt, ms)
    return out.reshape(NP, 16)[:n, 0]


# speedup vs baseline: 17.3972x; 1.1987x over previous
import functools

import jax
import jax.numpy as jnp
from jax import lax
from jax.experimental import pallas as pl
from jax.experimental.pallas import tpu as pltpu
from jax.experimental.pallas import tpu_sc as plsc

F32 = jnp.float32
I32 = jnp.int32

NC = 2
NS = 16
EB = 512


def _make_segsum(n_pad, epad, nq):
    NP = n_pad
    rows_node_pt = NP // NS
    ZB = rows_node_pt // 64
    n_zero = rows_node_pt // ZB
    if nq == 4:
        edges_pt = epad // NS
    else:
        edges_pt = epad // (NC * NS)
    nblocks = edges_pt // EB
    max_eb = epad - EB
    n_out = 4 if nq == 4 else NC

    mesh = plsc.VectorSubcoreMesh(core_axis_name="c", subcore_axis_name="s")

    @functools.partial(
        pl.kernel,
        out_type=jax.ShapeDtypeStruct((n_out, NP, 16), F32),
        mesh=mesh,
        scratch_types=[
            pltpu.VMEM((4, EB), I32),
            pltpu.VMEM((4, EB), I32),
            pltpu.VMEM((3, EB, 16), F32),
            pltpu.VMEM((ZB, 16), F32),
            pltpu.VMEM_SHARED((NP, 16), F32),
            pltpu.SemaphoreType.DMA,
            pltpu.SemaphoreType.DMA,
            pltpu.SemaphoreType.DMA,
        ],
        compiler_params=pltpu.CompilerParams(use_tc_tiling_on_sc=False),
    )
    def k(table, idx1, dst1, out, idx_v, dst_v, rows_v, zb, acc,
          lsem, gsem, ssem):
        c = lax.axis_index("c")
        s = lax.axis_index("s")

        def zb_body(i, carry):
            zb[i, :] = jnp.zeros((16,), F32)
            return carry

        lax.fori_loop(0, ZB, zb_body, 0)
        r0 = s * rows_node_pt

        def one_pass(q, edge_base):
            for z in range(n_zero):
                pltpu.async_copy(zb, acc.at[pl.ds(r0 + z * ZB, ZB)], gsem)
            for z in range(n_zero):
                pltpu.make_async_copy(
                    zb, acc.at[pl.ds(r0 + z * ZB, ZB)], gsem).wait()
            plsc.subcore_barrier()

            pltpu.async_copy(idx1.at[pl.ds(edge_base, EB)], idx_v.at[0], lsem)
            pltpu.async_copy(dst1.at[pl.ds(edge_base, EB)], dst_v.at[0], lsem)
            eb1 = jnp.minimum(edge_base + EB, max_eb)
            pltpu.async_copy(idx1.at[pl.ds(eb1, EB)], idx_v.at[1], lsem)
            pltpu.async_copy(dst1.at[pl.ds(eb1, EB)], dst_v.at[1], lsem)
            pltpu.make_async_copy(
                idx1.at[pl.ds(edge_base, EB)], idx_v.at[0], lsem).wait()
            pltpu.async_copy(
                table.at[q].at[idx_v.at[0]], rows_v.at[0], gsem)

            def block(i, carry):
                p = i % 3
                pn = (i + 1) % 3
                cp = i % 4
                cn = (i + 1) % 4
                cf = (i + 2) % 4
                eb_next = jnp.minimum(edge_base + (i + 2) * EB, max_eb)
                pltpu.make_async_copy(
                    idx1.at[pl.ds(edge_base, EB)], idx_v.at[cp], lsem).wait()
                pltpu.make_async_copy(
                    dst1.at[pl.ds(edge_base, EB)], dst_v.at[cp], lsem).wait()

                @pl.when(i < nblocks - 1)
                def _():
                    pltpu.async_copy(
                        table.at[q].at[idx_v.at[cn]], rows_v.at[pn], gsem)
                pltpu.async_copy(
                    idx1.at[pl.ds(eb_next, EB)], idx_v.at[cf], lsem)
                pltpu.async_copy(
                    dst1.at[pl.ds(eb_next, EB)], dst_v.at[cf], lsem)
                @pl.when(i > 0)
                def _():
                    pltpu.make_async_copy(
                        rows_v.at[0], acc.at[pl.ds(0, EB)], ssem).wait()
                pltpu.make_async_copy(
                    idx1.at[pl.ds(0, EB)], rows_v.at[p], gsem).wait()
                pltpu.async_copy(
                    rows_v.at[p], acc.at[dst_v.at[cp]], ssem, add=True)
                return carry

            lax.fori_loop(0, nblocks, block, 0)
            pltpu.make_async_copy(
                rows_v.at[0], acc.at[pl.ds(0, EB)], ssem).wait()
            for _ in range(3):
                pltpu.make_async_copy(
                    idx1.at[pl.ds(0, EB)], idx_v.at[0], lsem).wait()
            plsc.subcore_barrier()

        if nq == 4:
            for qq in range(2):
                q = c * 2 + qq
                one_pass(q, s * edges_pt)
                pltpu.sync_copy(
                    acc.at[pl.ds(r0, rows_node_pt)],
                    out.at[q, pl.ds(r0, rows_node_pt)],
                )
                if qq == 0:
                    plsc.subcore_barrier()
        else:
            one_pass(0, (c * NS + s) * edges_pt)
            pltpu.sync_copy(
                acc.at[pl.ds(r0, rows_node_pt)],
                out.at[c, pl.ds(r0, rows_node_pt)],
            )

    return k


def _dot(a, b):
    return jnp.dot(a, b, preferred_element_type=F32)


def _tc_layer0(n_pad, bn):
    grid = (n_pad // bn,)
    br = bn // 8

    def body(accp, x8, wn, ws, b, s6, h_out, cinv_out):
        a8 = accp[0] + accp[1]
        cnt = _dot(a8, s6[...])
        cinv = 1.0 / jnp.maximum(cnt, 1.0)
        z = _dot(a8 * cinv, wn[...]) + _dot(x8[...], ws[...]) + b[...]
        hc = jnp.maximum(z, 0.0)
        for qq in range(4):
            h_out[qq, :, :] = hc[:, 128 * qq:128 * (qq + 1)]
        cinv_out[...] = cinv

    return pl.pallas_call(
        body,
        grid=grid,
        in_specs=[
            pl.BlockSpec((2, br, 128), lambda i: (0, i, 0)),
            pl.BlockSpec((br, 128), lambda i: (i, 0)),
            pl.BlockSpec((128, 512), lambda i: (0, 0)),
            pl.BlockSpec((128, 512), lambda i: (0, 0)),
            pl.BlockSpec((1, 512), lambda i: (0, 0)),
            pl.BlockSpec((128, 128), lambda i: (0, 0)),
        ],
        out_specs=[
            pl.BlockSpec((4, br, 128), lambda i: (0, i, 0)),
            pl.BlockSpec((br, 128), lambda i: (i, 0)),
        ],
        out_shape=[
            jax.ShapeDtypeStruct((4, n_pad // 8, 128), F32),
            jax.ShapeDtypeStruct((n_pad // 8, 128), F32),
        ],
    )


def _tc_mid(n_pad, bn, last):
    grid = (n_pad // bn,)
    br = bn // 8

    def body(agg, h, cinv, wn, ws, b, *rest):
        if last:
            w5, b5, out = rest
        else:
            (out,) = rest
        x = jnp.concatenate([agg[qq] for qq in range(4)], axis=1)
        hh = jnp.concatenate([h[qq] for qq in range(4)], axis=1)
        ci = jnp.concatenate([cinv[...]] * 4, axis=1)
        z = _dot(x * ci, wn[...]) + _dot(hh, ws[...]) + b[...]
        hn = jnp.maximum(z, 0.0) + hh
        if last:
            out[...] = _dot(hn, w5[...]) + b5[...]
        else:
            for qq in range(4):
                out[qq, :, :] = hn[:, 128 * qq:128 * (qq + 1)]

    in_specs = [
        pl.BlockSpec((4, br, 128), lambda i: (0, i, 0)),
        pl.BlockSpec((4, br, 128), lambda i: (0, i, 0)),
        pl.BlockSpec((br, 128), lambda i: (i, 0)),
        pl.BlockSpec((512, 512), lambda i: (0, 0)),
        pl.BlockSpec((512, 512), lambda i: (0, 0)),
        pl.BlockSpec((1, 512), lambda i: (0, 0)),
    ]
    if last:
        in_specs += [
            pl.BlockSpec((512, 128), lambda i: (0, 0)),
            pl.BlockSpec((1, 128), lambda i: (0, 0)),
        ]
        out_spec = pl.BlockSpec((br, 128), lambda i: (i, 0))
        out_shape = jax.ShapeDtypeStruct((n_pad // 8, 128), F32)
    else:
        out_spec = pl.BlockSpec((4, br, 128), lambda i: (0, i, 0))
        out_shape = jax.ShapeDtypeStruct((4, n_pad // 8, 128), F32)

    return pl.pallas_call(
        body, grid=grid, in_specs=in_specs, out_specs=out_spec,
        out_shape=out_shape,
    )


def _tc_final(n_pad, bn):
    grid = (n_pad // bn,)
    br = bn // 8

    def body(accp, ts, cinv, mt, ms, out):
        op = accp[0] + accp[1]
        out[...] = _dot(op * cinv[...], mt[...]) + _dot(ts[...], ms[...])

    return pl.pallas_call(
        body,
        grid=grid,
        in_specs=[
            pl.BlockSpec((2, br, 128), lambda i: (0, i, 0)),
            pl.BlockSpec((br, 128), lambda i: (i, 0)),
            pl.BlockSpec((br, 128), lambda i: (i, 0)),
            pl.BlockSpec((128, 128), lambda i: (0, 0)),
            pl.BlockSpec((128, 128), lambda i: (0, 0)),
        ],
        out_specs=pl.BlockSpec((br, 128), lambda i: (i, 0)),
        out_shape=jax.ShapeDtypeStruct((n_pad // 8, 128), F32),
    )


def kernel(x, edge_index, Wn0, Ws0, b0, Wn1, Ws1, b1, Wn2, Ws2, b2,
           Wn3, Ws3, b3, Wn4, Ws4, b4, Wn5, Ws5, b5):
    n, in_dim = x.shape
    e = edge_index.shape[1]

    bn = 2048
    NP = -(-n // (NS * 8)) * (NS * 8)
    while NP % bn:
        NP += NS * 8

    src = edge_index[0].astype(I32)
    dst = edge_index[1].astype(I32)
    unit = NC * NS * EB
    epad = ((e + unit - 1) // unit) * unit
    pad = epad - e
    src_p = jnp.concatenate([src, jnp.zeros((pad,), I32)])
    dst_p = jnp.concatenate([dst, jnp.full((pad,), n, I32)])

    x16 = jnp.concatenate(
        [x, jnp.ones((n, 1), F32), jnp.zeros((n, 16 - in_dim - 1), F32)], 1)
    x16 = jnp.concatenate([x16, jnp.zeros((NP - n, 16), F32)], 0)
    x8 = x16.reshape(NP // 8, 128)

    eye8 = jnp.eye(8, dtype=F32)
    wn0p = jnp.zeros((16, 64), F32).at[:in_dim].set(Wn0)
    ws0p = jnp.zeros((16, 64), F32).at[:in_dim].set(Ws0)

    def big0(w):
        w4 = w.reshape(16, 4, 16)
        return jnp.einsum('cpd,jk->jcpkd', w4, eye8).reshape(128, 512)

    def big(w):
        w4 = w.reshape(4, 16, 4, 16)
        return jnp.einsum('qcpd,jk->qjcpkd', w4, eye8).reshape(512, 512)

    def bpack(b):
        return jnp.tile(b.reshape(4, 1, 16), (1, 8, 1)).reshape(1, 512)

    s6 = jnp.kron(eye8, jnp.zeros((16, 16), F32).at[6, :].set(1.0))
    mt = jnp.kron(eye8, jnp.zeros((16, 16), F32).at[0, 0].set(1.0))
    ms = jnp.kron(eye8, jnp.zeros((16, 16), F32).at[1, 0].set(1.0))

    w5p = jnp.zeros((64, 16), F32).at[:, 0:1].set(Wn5).at[:, 1:2].set(Ws5)
    w5big = jnp.einsum('qcd,jk->qjckd', w5p.reshape(4, 16, 16),
                       eye8).reshape(512, 128)
    b5a = jnp.tile(jnp.zeros((1, 16), F32).at[0, 1].set(b5[0]),
                   (1, 8)).reshape(1, 128)

    seg_single = _make_segsum(NP, epad, nq=1)
    seg_quad = _make_segsum(NP, epad, nq=4)

    def to128(a):
        return a.reshape(a.shape[0], NP // 8, 128)

    accp0 = to128(seg_single(x16.reshape(1, NP, 16), src_p, dst_p))
    h, cinv = _tc_layer0(NP, bn)(
        accp0, x8, big0(wn0p), big0(ws0p), bpack(b0), s6)

    for (wn, ws, b) in ((Wn1, Ws1, b1), (Wn2, Ws2, b2), (Wn3, Ws3, b3)):
        agg = to128(seg_quad(h.reshape(4, NP, 16), src_p, dst_p))
        h = _tc_mid(NP, bn, last=False)(
            agg, h, cinv, big(wn), big(ws), bpack(b))

    agg = to128(seg_quad(h.reshape(4, NP, 16), src_p, dst_p))
    ts = _tc_mid(NP, bn, last=True)(
        agg, h, cinv, big(Wn4), big(Ws4), bpack(b4), w5big, b5a)

    accp5 = to128(seg_single(ts.reshape(1, NP, 16), src_p, dst_p))
    out = _tc_final(NP, bn)(accp5, ts, cinv, mt, ms)
    return out.reshape(NP, 16)[:n, 0]
